# baseline (speedup 1.0000x reference)
import jax
import jax.numpy as jnp
from jax import lax
from jax.experimental import pallas as pl
from jax.experimental.pallas import tpu as pltpu
from jax.experimental.pallas import tpu_sc as plsc

_NUM_EMB = 1000000
_DIM = 64
_BATCH = 4096
_SEQ = 200

_NC = 2
_NS = 16
_NW = _NC * _NS
_BPW = _BATCH // _NW
_L = 16


def _iota16():
  return lax.iota(jnp.int32, _L)


def _body(idx_hbm, table_hbm, out_hbm, idx_v, idx_t, buf_a, buf_b,
          tbuf_a, tbuf_b, sem_ga, sem_gb, sem_sa, sem_sb):
  wid = lax.axis_index("s") * _NC + lax.axis_index("c")
  b0 = wid * _BPW
  pltpu.sync_copy(idx_hbm.at[pl.ds(b0, _BPW)], idx_v)

  iota = _iota16()
  rows_lb = [iota + lb * _L for lb in range(_BPW // _L)]

  @pl.loop(0, _SEQ)
  def _build(t):
    col = jnp.full((_L,), t, jnp.int32)
    for lb in range(_BPW // _L):
      v = plsc.load_gather(idx_v, [rows_lb[lb], col])
      idx_t[t, pl.ds(lb * _L, _L)] = v

  frow = [(iota + k * _L) // 8 for k in range(_DIM // _L)]
  fsub = [(iota + k * _L) % 8 for k in range(_DIM // _L)]

  def transpose(buf, tbuf):
    @pl.loop(0, _BPW, unroll=8)
    def _row(r):
      lane = jnp.full((_L,), r, jnp.int32)
      for k in range(_DIM // _L):
        v = buf[r, pl.ds(k * _L, _L)]
        plsc.store_scatter(tbuf, [frow[k], fsub[k], lane], v)

  def wait_store(tbuf, sem):
    pltpu.make_async_copy(tbuf, out_hbm.at[0, :, 0], sem).wait()

  @pl.loop(0, _SEQ // 2)
  def _pair(i):
    @pl.when(i > 0)
    def _():
      wait_store(tbuf_a, sem_sa)
      wait_store(tbuf_b, sem_sb)
    ga = pltpu.async_copy(table_hbm.at[idx_t.at[2 * i]], buf_a, sem_ga)
    gb = pltpu.async_copy(table_hbm.at[idx_t.at[2 * i + 1]], buf_b, sem_gb)
    ga.wait()
    transpose(buf_a, tbuf_a)
    pltpu.async_copy(tbuf_a, out_hbm.at[2 * i, :, wid], sem_sa)
    gb.wait()
    transpose(buf_b, tbuf_b)
    pltpu.async_copy(tbuf_b, out_hbm.at[2 * i + 1, :, wid], sem_sb)

  wait_store(tbuf_a, sem_sa)
  wait_store(tbuf_b, sem_sb)


def kernel(x, weight):
  mesh = plsc.VectorSubcoreMesh(
      core_axis_name="c", subcore_axis_name="s",
      num_cores=_NC, num_subcores=_NS)
  o5 = pl.kernel(
      _body,
      out_type=jax.ShapeDtypeStruct((_SEQ, 8, _NW, 8, 128), jnp.float32),
      mesh=mesh,
      scratch_types=[
          pltpu.VMEM((_BPW, _SEQ), jnp.int32),
          pltpu.VMEM((_SEQ, _BPW), jnp.int32),
          pltpu.VMEM((_BPW, _DIM), jnp.float32),
          pltpu.VMEM((_BPW, _DIM), jnp.float32),
          pltpu.VMEM((8, 8, 128), jnp.float32),
          pltpu.VMEM((8, 8, 128), jnp.float32),
          pltpu.SemaphoreType.DMA,
          pltpu.SemaphoreType.DMA,
          pltpu.SemaphoreType.DMA,
          pltpu.SemaphoreType.DMA,
      ],
      compiler_params=pltpu.CompilerParams(
          use_tc_tiling_on_sc=False, needs_layout_passes=False),
  )(x.astype(jnp.int32), weight)
  return o5.transpose(2, 4, 0, 1, 3).reshape(_BATCH, _SEQ, _DIM)


# speedup vs baseline: 1.4075x; 1.4075x over previous
import jax
import jax.numpy as jnp
from jax import lax
from jax.experimental import pallas as pl
from jax.experimental.pallas import tpu as pltpu
from jax.experimental.pallas import tpu_sc as plsc

_NUM_EMB = 1000000
_DIM = 64
_BATCH = 4096
_SEQ = 200

_NC = 2
_NS = 16
_NW = _NC * _NS

_TOTAL = _BATCH * _SEQ
_PER_W = _TOTAL // _NW
_CHUNK = 128
_NCHUNK = _PER_W // _CHUNK
_K = 4
_NGROUP = _NCHUNK // _K
_NPAIR = _NGROUP // 2


def _body(idx_hbm, table_hbm, out_hbm, idx_v, rows_a, rows_b,
          sem_ga, sem_gb, sem_sa, sem_sb):
  wid = lax.axis_index("s") * _NC + lax.axis_index("c")
  pltpu.sync_copy(idx_hbm.at[wid], idx_v)

  row0 = wid * _PER_W

  def fire_gathers(g, rows, sem):
    return [
        pltpu.async_copy(table_hbm.at[idx_v.at[g * _K + b]], rows.at[b], sem)
        for b in range(_K)
    ]

  def fire_stores(g, rows, sem):
    for b in range(_K):
      pltpu.async_copy(
          rows.at[b],
          out_hbm.at[pl.ds(row0 + (g * _K + b) * _CHUNK, _CHUNK)], sem)

  def wait_stores(rows, sem):
    for b in range(_K):
      pltpu.make_async_copy(
          rows.at[b], out_hbm.at[pl.ds(row0, _CHUNK)], sem).wait()

  @pl.loop(0, _NPAIR)
  def _pair(i):
    @pl.when(i > 0)
    def _():
      wait_stores(rows_a, sem_sa)
      wait_stores(rows_b, sem_sb)
    ga = fire_gathers(2 * i, rows_a, sem_ga)
    gb = fire_gathers(2 * i + 1, rows_b, sem_gb)
    for cp in ga:
      cp.wait()
    fire_stores(2 * i, rows_a, sem_sa)
    for cp in gb:
      cp.wait()
    fire_stores(2 * i + 1, rows_b, sem_sb)

  wait_stores(rows_a, sem_sa)
  wait_stores(rows_b, sem_sb)


def kernel(x, weight):
  idx = x.astype(jnp.int32).reshape(_NW, _NCHUNK, _CHUNK)
  mesh = plsc.VectorSubcoreMesh(
      core_axis_name="c", subcore_axis_name="s",
      num_cores=_NC, num_subcores=_NS)
  out = pl.kernel(
      _body,
      out_type=jax.ShapeDtypeStruct((_TOTAL, _DIM), jnp.float32),
      mesh=mesh,
      scratch_types=[
          pltpu.VMEM((_NCHUNK, _CHUNK), jnp.int32),
          pltpu.VMEM((_K, _CHUNK, _DIM), jnp.float32),
          pltpu.VMEM((_K, _CHUNK, _DIM), jnp.float32),
          pltpu.SemaphoreType.DMA,
          pltpu.SemaphoreType.DMA,
          pltpu.SemaphoreType.DMA,
          pltpu.SemaphoreType.DMA,
      ],
      compiler_params=pltpu.CompilerParams(use_tc_tiling_on_sc=False),
  )(idx, weight)
  return out.reshape(_BATCH, _SEQ, _DIM)


# speedup vs baseline: 1.4104x; 1.0020x over previous
import jax
import jax.numpy as jnp
from jax import lax
from jax.experimental import pallas as pl
from jax.experimental.pallas import tpu as pltpu
from jax.experimental.pallas import tpu_sc as plsc

_NUM_EMB = 1000000
_DIM = 64
_BATCH = 4096
_SEQ = 200

_NC = 2
_NS = 16
_NW = _NC * _NS

_TOTAL = _BATCH * _SEQ
_PER_W = _TOTAL // _NW
_CHUNK = 128
_NCHUNK = _PER_W // _CHUNK
_K = 5
_NGROUP = _NCHUNK // _K
_NPAIR = _NGROUP // 2


def _body(idx_hbm, table_hbm, out_hbm, idx_v, rows_a, rows_b,
          sem_ga, sem_gb, sem_sa, sem_sb):
  wid = lax.axis_index("s") * _NC + lax.axis_index("c")
  pltpu.sync_copy(idx_hbm.at[wid], idx_v)

  row0 = wid * _PER_W

  def fire_gathers(g, rows, sem):
    return [
        pltpu.async_copy(table_hbm.at[idx_v.at[g * _K + b]], rows.at[b], sem)
        for b in range(_K)
    ]

  def fire_stores(g, rows, sem):
    for b in range(_K):
      pltpu.async_copy(
          rows.at[b],
          out_hbm.at[pl.ds(row0 + (g * _K + b) * _CHUNK, _CHUNK)], sem)

  def wait_stores(rows, sem):
    for b in range(_K):
      pltpu.make_async_copy(
          rows.at[b], out_hbm.at[pl.ds(row0, _CHUNK)], sem).wait()

  @pl.loop(0, _NPAIR)
  def _pair(i):
    @pl.when(i > 0)
    def _():
      wait_stores(rows_a, sem_sa)
      wait_stores(rows_b, sem_sb)
    ga = fire_gathers(2 * i, rows_a, sem_ga)
    gb = fire_gathers(2 * i + 1, rows_b, sem_gb)
    for cp in ga:
      cp.wait()
    fire_stores(2 * i, rows_a, sem_sa)
    for cp in gb:
      cp.wait()
    fire_stores(2 * i + 1, rows_b, sem_sb)

  wait_stores(rows_a, sem_sa)
  wait_stores(rows_b, sem_sb)


def kernel(x, weight):
  idx = x.astype(jnp.int32).reshape(_NW, _NCHUNK, _CHUNK)
  mesh = plsc.VectorSubcoreMesh(
      core_axis_name="c", subcore_axis_name="s",
      num_cores=_NC, num_subcores=_NS)
  out = pl.kernel(
      _body,
      out_type=jax.ShapeDtypeStruct((_TOTAL, _DIM), jnp.float32),
      mesh=mesh,
      scratch_types=[
          pltpu.VMEM((_NCHUNK, _CHUNK), jnp.int32),
          pltpu.VMEM((_K, _CHUNK, _DIM), jnp.float32),
          pltpu.VMEM((_K, _CHUNK, _DIM), jnp.float32),
          pltpu.SemaphoreType.DMA,
          pltpu.SemaphoreType.DMA,
          pltpu.SemaphoreType.DMA,
          pltpu.SemaphoreType.DMA,
      ],
      compiler_params=pltpu.CompilerParams(use_tc_tiling_on_sc=False),
  )(idx, weight)
  return out.reshape(_BATCH, _SEQ, _DIM)
